# Initial kernel scaffold; baseline (speedup 1.0000x reference)
import jax
import jax.numpy as jnp
from jax.experimental import pallas as pl


def kernel(x, revin_w, revin_b, gate_W1, gate_W2, Ws_s, bs_s, Ws_t, bs_t):
    raise NotImplementedError("write your pallas kernel here")


# speedup vs baseline: 2.0859x; 2.0859x over previous
import functools

import jax
import jax.numpy as jnp
import numpy as np
from jax.experimental import pallas as pl
from jax.experimental.pallas import tpu as pltpu

B, L, C, D, E, H, K, KER = 512, 336, 21, 256, 8, 256, 2, 25


def _build_ma_matrix() -> np.ndarray:
    pad = (KER - 1) // 2
    M = np.zeros((L, L), np.float32)
    for l in range(L):
        for j in range(l - pad, l + pad + 1):
            M[l, min(max(j, 0), L - 1)] += 1.0
    return M / KER


_MA = _build_ma_matrix()


def _weff_body(ws_s_ref, ws_t_ref, ma_ref, weff_ref):
    ws = ws_s_ref[0]
    wt = ws_t_ref[0]
    weff_ref[0] = ws + jnp.dot(wt - ws, ma_ref[...],
                               preferred_element_type=jnp.float32)


def _fold_weights(ws_s, ws_t, ma):
    return pl.pallas_call(
        _weff_body,
        grid=(E,),
        in_specs=[
            pl.BlockSpec((1, D, L), lambda e: (e, 0, 0)),
            pl.BlockSpec((1, D, L), lambda e: (e, 0, 0)),
            pl.BlockSpec((L, L), lambda e: (0, 0)),
        ],
        out_specs=pl.BlockSpec((1, D, L), lambda e: (e, 0, 0)),
        out_shape=jax.ShapeDtypeStruct((E, D, L), jnp.float32),
    )(ws_s, ws_t, ma)


def _revin_gate_body(xt_ref, rw_ref, rb_ref, w1_ref, w2_ref,
                     xn_ref, gates_ref):
    xt = xt_ref[...]
    mu = jnp.mean(xt, axis=2, keepdims=True)
    var = jnp.mean((xt - mu) ** 2, axis=2, keepdims=True)
    sd = jnp.sqrt(var + 1e-5)
    rw = rw_ref[...].reshape(1, C, 1)
    rb = rb_ref[...].reshape(1, C, 1)
    xn = (xt - mu) / sd * rw + rb
    xn_ref[...] = xn

    m = jnp.mean(xn, axis=1)
    h = jax.nn.relu(jnp.dot(m, w1_ref[...],
                            preferred_element_type=jnp.float32))
    logits = jnp.dot(h, w2_ref[...],
                     preferred_element_type=jnp.float32)
    lmax = jnp.max(logits, axis=1, keepdims=True)
    ex = jnp.exp(logits - lmax)
    p = ex / jnp.sum(ex, axis=1, keepdims=True)
    iota = jax.lax.broadcasted_iota(jnp.int32, p.shape, 1)
    v1 = jnp.max(p, axis=1, keepdims=True)
    i1 = jnp.min(jnp.where(p == v1, iota, E), axis=1, keepdims=True)
    p2 = jnp.where(iota == i1, -jnp.inf, p)
    v2 = jnp.max(p2, axis=1, keepdims=True)
    i2 = jnp.min(jnp.where(p2 == v2, iota, E), axis=1, keepdims=True)
    denom = v1 + v2 + 1e-6
    g1 = v1 / denom
    g2 = v2 / denom
    gates_ref[...] = (jnp.where(iota == i1, g1, 0.0)
                      + jnp.where(iota == i2, g2, 0.0))


def _revin_gate(xt, rw, rb, w1, w2, bb):
    nb = B // bb
    return pl.pallas_call(
        _revin_gate_body,
        grid=(nb,),
        in_specs=[
            pl.BlockSpec((bb, C, L), lambda i: (i, 0, 0)),
            pl.BlockSpec((C,), lambda i: (0,)),
            pl.BlockSpec((C,), lambda i: (0,)),
            pl.BlockSpec((L, H), lambda i: (0, 0)),
            pl.BlockSpec((H, E), lambda i: (0, 0)),
        ],
        out_specs=[
            pl.BlockSpec((bb, C, L), lambda i: (i, 0, 0)),
            pl.BlockSpec((bb, E), lambda i: (i, 0)),
        ],
        out_shape=[
            jax.ShapeDtypeStruct((B, C, L), jnp.float32),
            jax.ShapeDtypeStruct((B, E), jnp.float32),
        ],
    )(xt, rw, rb, w1, w2)


def _loss_body(gates_ref, out_ref):
    g = gates_ref[...]
    importance = jnp.sum(g, axis=0)
    load = jnp.sum((g > 0).astype(jnp.float32), axis=0)

    def cv2(v):
        mean = jnp.mean(v)
        var = jnp.sum((v - mean) ** 2) / (E - 1)
        return var / (mean ** 2 + 1e-10)

    out_ref[0, 0] = cv2(importance) + cv2(load)


def _loss(gates):
    return pl.pallas_call(
        _loss_body,
        out_shape=jax.ShapeDtypeStruct((1, 1), jnp.float32),
        out_specs=pl.BlockSpec(memory_space=pltpu.SMEM),
    )(gates)


def _moe_body(xn_ref, grow_ref, weff_ref, bs_ref, y_ref):
    e = pl.program_id(1)
    iota = jax.lax.broadcasted_iota(jnp.int32, (grow_ref.shape[0], E), 1)
    g = jnp.sum(jnp.where(iota == e, grow_ref[...], 0.0),
                axis=1, keepdims=True)
    acc = jax.lax.dot_general(
        xn_ref[...], weff_ref[0],
        dimension_numbers=(((1,), (1,)), ((), ())),
        preferred_element_type=jnp.float32)
    contrib = g * (acc + bs_ref[...][0:1, :])

    @pl.when(e == 0)
    def _():
        y_ref[...] = contrib

    @pl.when(e > 0)
    def _():
        y_ref[...] += contrib


def _moe_matmul(xn_rows, gates_rows, weff, bias, rb):
    nrb = (B * C) // rb
    return pl.pallas_call(
        _moe_body,
        grid=(nrb, E),
        in_specs=[
            pl.BlockSpec((rb, L), lambda i, e: (i, 0)),
            pl.BlockSpec((rb, E), lambda i, e: (i, 0)),
            pl.BlockSpec((1, D, L), lambda i, e: (e, 0, 0)),
            pl.BlockSpec((E, D), lambda i, e: (0, 0)),
        ],
        out_specs=pl.BlockSpec((rb, D), lambda i, e: (i, 0)),
        out_shape=jax.ShapeDtypeStruct((B * C, D), jnp.float32),
    )(xn_rows, gates_rows, weff, bias)


def kernel(x, revin_w, revin_b, gate_W1, gate_W2, Ws_s, bs_s, Ws_t, bs_t):
    xt = jnp.transpose(x, (0, 2, 1))
    ma = jnp.asarray(_MA)
    weff = _fold_weights(Ws_s, Ws_t, ma)
    xn_t, gates = _revin_gate(xt, revin_w, revin_b, gate_W1, gate_W2, 64)
    loss = _loss(gates)[0, 0]
    bias = bs_s + bs_t
    xn_rows = xn_t.reshape(B * C, L)
    gates_rows = jnp.repeat(gates, C, axis=0)
    y = _moe_matmul(xn_rows, gates_rows, weff, bias, 1344)
    return y.reshape(B, C, D), loss
